# baseline (speedup 1.0000x reference)
import numpy as np
import jax
import jax.numpy as jnp
from jax.experimental import pallas as pl
from jax.experimental.pallas import tpu as pltpu

MUL = 128
INV_SQRT3_ = 0.5773502691896258
BLOCK = 1000

_T4 = np.kron(np.eye(4, dtype=np.float32), np.ones((1, MUL), np.float32))


def _body(x0_ref, x1_ref, yt_ref, t4_ref, w0_ref, w1_ref,
          b_ref, o_ref):
    bf16 = jnp.bfloat16
    f32 = jnp.float32
    yy = jax.lax.dot_general(
        yt_ref[0].astype(bf16), t4_ref[...],
        (((0,), (0,)), ((), ())),
        preferred_element_type=f32).astype(bf16)
    y0 = yy[:, :MUL]
    y1 = [yy[:, MUL:2 * MUL], yy[:, 2 * MUL:3 * MUL], yy[:, 3 * MUL:]]

    x0 = x0_ref[...].astype(bf16)
    x1 = [x1_ref[0].astype(bf16), x1_ref[1].astype(bf16),
          x1_ref[2].astype(bf16)]

    l0 = jnp.concatenate(
        [x0 * y0, x1[0] * y1[0], x1[1] * y1[1], x1[2] * y1[2]], axis=1
    )
    o_ref[:, :MUL] = (
        jnp.dot(l0, w0_ref[...], preferred_element_type=f32) + b_ref[...]
    )

    l1 = jnp.concatenate(
        [x0 * y1[0], x0 * y1[1], x0 * y1[2],
         y0 * x1[0], y0 * x1[1], y0 * x1[2]], axis=1
    )
    o_ref[:, MUL:] = jnp.dot(l1, w1_ref[...], preferred_element_type=f32)


def _perm_kron(w):
    eye3 = jnp.eye(3, dtype=w.dtype)
    return jnp.einsum("ij,uw->iuwj", eye3, w).reshape(3 * MUL, 3 * MUL)


def kernel(x_0e, x_1o, y_0e, y_1o, w_ss, w_vv, w_sv, w_vs, b):
    n = x_0e.shape[0]
    x1t = jnp.transpose(x_1o, (2, 0, 1))
    yt = jnp.concatenate([y_0e.T, y_1o.T], axis=0)
    ytr = yt.reshape(4, n // BLOCK, BLOCK).transpose(1, 0, 2)

    bf16 = jnp.bfloat16
    t4 = jnp.asarray(_T4, dtype=bf16)
    wvv = w_vv * INV_SQRT3_
    w0 = jnp.concatenate([w_ss, wvv, wvv, wvv], axis=0).astype(bf16)
    w1 = jnp.concatenate(
        [_perm_kron(w_sv), _perm_kron(w_vs)], axis=0
    ).astype(bf16)
    b2 = b.reshape(1, MUL)

    grid = n // BLOCK
    row_spec = lambda width: pl.BlockSpec((BLOCK, width), lambda i: (i, 0))
    full_spec = lambda a: pl.BlockSpec(a.shape, lambda i: (0, 0))

    return pl.pallas_call(
        _body,
        grid=(grid,),
        in_specs=[
            row_spec(MUL),
            pl.BlockSpec((3, BLOCK, MUL), lambda i: (0, i, 0)),
            pl.BlockSpec((1, 4, BLOCK), lambda i: (i, 0, 0)),
            full_spec(t4),
            full_spec(w0),
            full_spec(w1),
            full_spec(b2),
        ],
        out_specs=row_spec(MUL * 4),
        out_shape=jax.ShapeDtypeStruct((n, MUL * 4), jnp.float32),
        compiler_params=pltpu.CompilerParams(
            dimension_semantics=("arbitrary",),
            vmem_limit_bytes=50 * 1024 * 1024,
        ),
    )(x_0e, x1t, ytr, t4, w0, w1, b2)


# speedup vs baseline: 1.2731x; 1.2731x over previous
import numpy as np
import jax
import jax.numpy as jnp
from jax.experimental import pallas as pl
from jax.experimental.pallas import tpu as pltpu

MUL = 128
INV_SQRT3_ = 0.5773502691896258
BLOCK = 4000

_T4 = np.kron(np.eye(4, dtype=np.float32), np.ones((1, MUL), np.float32))


def _body(x0_ref, x1_ref, yt_ref, t4_ref, w0_ref, w1_ref,
          b_ref, o_ref):
    bf16 = jnp.bfloat16
    f32 = jnp.float32
    yy = jax.lax.dot_general(
        yt_ref[0].astype(bf16), t4_ref[...],
        (((0,), (0,)), ((), ())),
        preferred_element_type=f32).astype(bf16)
    y0 = yy[:, :MUL]
    y1 = [yy[:, MUL:2 * MUL], yy[:, 2 * MUL:3 * MUL], yy[:, 3 * MUL:]]

    x0 = x0_ref[...].astype(bf16)
    x1 = [x1_ref[0].astype(bf16), x1_ref[1].astype(bf16),
          x1_ref[2].astype(bf16)]

    l0 = jnp.concatenate(
        [x0 * y0, x1[0] * y1[0], x1[1] * y1[1], x1[2] * y1[2]], axis=1
    )
    o_ref[:, :MUL] = (
        jnp.dot(l0, w0_ref[...], preferred_element_type=f32) + b_ref[...]
    )

    l1 = jnp.concatenate(
        [x0 * y1[0], x0 * y1[1], x0 * y1[2],
         y0 * x1[0], y0 * x1[1], y0 * x1[2]], axis=1
    )
    o_ref[:, MUL:] = jnp.dot(l1, w1_ref[...], preferred_element_type=f32)


def _perm_kron(w):
    eye3 = jnp.eye(3, dtype=w.dtype)
    return jnp.einsum("ij,uw->iuwj", eye3, w).reshape(3 * MUL, 3 * MUL)


def kernel(x_0e, x_1o, y_0e, y_1o, w_ss, w_vv, w_sv, w_vs, b):
    n = x_0e.shape[0]
    x1t = jnp.transpose(x_1o, (2, 0, 1))
    yt = jnp.concatenate([y_0e.T, y_1o.T], axis=0)
    ytr = yt.reshape(4, n // BLOCK, BLOCK).transpose(1, 0, 2)

    bf16 = jnp.bfloat16
    t4 = jnp.asarray(_T4, dtype=bf16)
    wvv = w_vv * INV_SQRT3_
    w0 = jnp.concatenate([w_ss, wvv, wvv, wvv], axis=0).astype(bf16)
    w1 = jnp.concatenate(
        [_perm_kron(w_sv), _perm_kron(w_vs)], axis=0
    ).astype(bf16)
    b2 = b.reshape(1, MUL)

    grid = n // BLOCK
    row_spec = lambda width: pl.BlockSpec((BLOCK, width), lambda i: (i, 0))
    full_spec = lambda a: pl.BlockSpec(a.shape, lambda i: (0, 0))

    return pl.pallas_call(
        _body,
        grid=(grid,),
        in_specs=[
            row_spec(MUL),
            pl.BlockSpec((3, BLOCK, MUL), lambda i: (0, i, 0)),
            pl.BlockSpec((1, 4, BLOCK), lambda i: (i, 0, 0)),
            full_spec(t4),
            full_spec(w0),
            full_spec(w1),
            full_spec(b2),
        ],
        out_specs=row_spec(MUL * 4),
        out_shape=jax.ShapeDtypeStruct((n, MUL * 4), jnp.float32),
        compiler_params=pltpu.CompilerParams(
            dimension_semantics=("arbitrary",),
            vmem_limit_bytes=56 * 1024 * 1024,
        ),
    )(x_0e, x1t, ytr, t4, w0, w1, b2)


# speedup vs baseline: 1.2919x; 1.0148x over previous
import numpy as np
import jax
import jax.numpy as jnp
from jax.experimental import pallas as pl
from jax.experimental.pallas import tpu as pltpu

MUL = 128
INV_SQRT3_ = 0.5773502691896258
BLOCK = 5000

_T4 = np.kron(np.eye(4, dtype=np.float32), np.ones((1, MUL), np.float32))


def _body(x0_ref, x1_ref, yt_ref, t4_ref, w0_ref, w1_ref,
          b_ref, o_ref):
    bf16 = jnp.bfloat16
    f32 = jnp.float32
    yy = jax.lax.dot_general(
        yt_ref[0].astype(bf16), t4_ref[...],
        (((0,), (0,)), ((), ())),
        preferred_element_type=f32).astype(bf16)
    y0 = yy[:, :MUL]
    y1 = [yy[:, MUL:2 * MUL], yy[:, 2 * MUL:3 * MUL], yy[:, 3 * MUL:]]

    x0 = x0_ref[...].astype(bf16)
    x1 = [x1_ref[0].astype(bf16), x1_ref[1].astype(bf16),
          x1_ref[2].astype(bf16)]

    l0 = jnp.concatenate(
        [x0 * y0, x1[0] * y1[0], x1[1] * y1[1], x1[2] * y1[2]], axis=1
    )
    o_ref[:, :MUL] = (
        jnp.dot(l0, w0_ref[...], preferred_element_type=f32) + b_ref[...]
    )

    l1 = jnp.concatenate(
        [x0 * y1[0], x0 * y1[1], x0 * y1[2],
         y0 * x1[0], y0 * x1[1], y0 * x1[2]], axis=1
    )
    o_ref[:, MUL:] = jnp.dot(l1, w1_ref[...], preferred_element_type=f32)


def _perm_kron(w):
    eye3 = jnp.eye(3, dtype=w.dtype)
    return jnp.einsum("ij,uw->iuwj", eye3, w).reshape(3 * MUL, 3 * MUL)


def kernel(x_0e, x_1o, y_0e, y_1o, w_ss, w_vv, w_sv, w_vs, b):
    n = x_0e.shape[0]
    x1t = jnp.transpose(x_1o, (2, 0, 1))
    yt = jnp.concatenate([y_0e.T, y_1o.T], axis=0)
    ytr = yt.reshape(4, n // BLOCK, BLOCK).transpose(1, 0, 2)

    bf16 = jnp.bfloat16
    t4 = jnp.asarray(_T4, dtype=bf16)
    wvv = w_vv * INV_SQRT3_
    w0 = jnp.concatenate([w_ss, wvv, wvv, wvv], axis=0).astype(bf16)
    w1 = jnp.concatenate(
        [_perm_kron(w_sv), _perm_kron(w_vs)], axis=0
    ).astype(bf16)
    b2 = b.reshape(1, MUL)

    grid = n // BLOCK
    row_spec = lambda width: pl.BlockSpec((BLOCK, width), lambda i: (i, 0))
    full_spec = lambda a: pl.BlockSpec(a.shape, lambda i: (0, 0))

    return pl.pallas_call(
        _body,
        grid=(grid,),
        in_specs=[
            row_spec(MUL),
            pl.BlockSpec((3, BLOCK, MUL), lambda i: (0, i, 0)),
            pl.BlockSpec((1, 4, BLOCK), lambda i: (i, 0, 0)),
            full_spec(t4),
            full_spec(w0),
            full_spec(w1),
            full_spec(b2),
        ],
        out_specs=row_spec(MUL * 4),
        out_shape=jax.ShapeDtypeStruct((n, MUL * 4), jnp.float32),
        compiler_params=pltpu.CompilerParams(
            dimension_semantics=("arbitrary",),
            vmem_limit_bytes=60 * 1024 * 1024,
        ),
    )(x_0e, x1t, ytr, t4, w0, w1, b2)


# speedup vs baseline: 1.3384x; 1.0360x over previous
import numpy as np
import jax
import jax.numpy as jnp
from jax.experimental import pallas as pl
from jax.experimental.pallas import tpu as pltpu

MUL = 128
INV_SQRT3_ = 0.5773502691896258
BLOCK = 5000

_T4 = np.kron(np.eye(4, dtype=np.float32), np.ones((1, MUL), np.float32))


def _body(x0_ref, x1_ref, yt_ref, t4_ref, wss_ref, wvv_ref, wsv_ref, wvs_ref,
          b_ref, o_ref, w0_s, w1_s):
    bf16 = jnp.bfloat16
    f32 = jnp.float32

    @pl.when(pl.program_id(0) == 0)
    def _prep():
        w0_s[:MUL, :] = wss_ref[...].astype(bf16)
        wvv = (wvv_ref[...] * INV_SQRT3_).astype(bf16)
        for k in range(3):
            w0_s[MUL * (k + 1):MUL * (k + 2), :] = wvv
        r = jax.lax.broadcasted_iota(jnp.int32, (MUL, 3 * MUL), 0)
        c = jax.lax.broadcasted_iota(jnp.int32, (MUL, 3 * MUL), 1)
        wsv = wsv_ref[...].astype(bf16)
        wvs = wvs_ref[...].astype(bf16)
        for i in range(3):
            e_i = jnp.where(c == 3 * r + i, 1.0, 0.0).astype(bf16)
            w1_s[MUL * i:MUL * (i + 1), :] = jnp.dot(
                wsv, e_i, preferred_element_type=f32).astype(bf16)
            w1_s[3 * MUL + MUL * i:3 * MUL + MUL * (i + 1), :] = jnp.dot(
                wvs, e_i, preferred_element_type=f32).astype(bf16)

    yy = jax.lax.dot_general(
        yt_ref[0].astype(bf16), t4_ref[...],
        (((0,), (0,)), ((), ())),
        preferred_element_type=f32).astype(bf16)
    y0 = yy[:, :MUL]
    y1 = [yy[:, MUL:2 * MUL], yy[:, 2 * MUL:3 * MUL], yy[:, 3 * MUL:]]

    x0 = x0_ref[...].astype(bf16)
    x1 = [x1_ref[0].astype(bf16), x1_ref[1].astype(bf16),
          x1_ref[2].astype(bf16)]

    l0 = jnp.concatenate(
        [x0 * y0, x1[0] * y1[0], x1[1] * y1[1], x1[2] * y1[2]], axis=1
    )
    o_ref[:, :MUL] = (
        jnp.dot(l0, w0_s[...], preferred_element_type=f32) + b_ref[...]
    )

    l1 = jnp.concatenate(
        [x0 * y1[0], x0 * y1[1], x0 * y1[2],
         y0 * x1[0], y0 * x1[1], y0 * x1[2]], axis=1
    )
    o_ref[:, MUL:] = jnp.dot(l1, w1_s[...], preferred_element_type=f32)


def kernel(x_0e, x_1o, y_0e, y_1o, w_ss, w_vv, w_sv, w_vs, b):
    n = x_0e.shape[0]
    x1t = jnp.transpose(x_1o, (2, 0, 1))
    yt = jnp.concatenate([y_0e.T, y_1o.T], axis=0)
    ytr = yt.reshape(4, n // BLOCK, BLOCK).transpose(1, 0, 2)

    t4 = jnp.asarray(_T4, dtype=jnp.bfloat16)
    b2 = b.reshape(1, MUL)

    grid = n // BLOCK
    row_spec = lambda width: pl.BlockSpec((BLOCK, width), lambda i: (i, 0))
    full_spec = lambda a: pl.BlockSpec(a.shape, lambda i: (0, 0))

    return pl.pallas_call(
        _body,
        grid=(grid,),
        in_specs=[
            row_spec(MUL),
            pl.BlockSpec((3, BLOCK, MUL), lambda i: (0, i, 0)),
            pl.BlockSpec((1, 4, BLOCK), lambda i: (i, 0, 0)),
            full_spec(t4),
            full_spec(w_ss),
            full_spec(w_vv),
            full_spec(w_sv),
            full_spec(w_vs),
            full_spec(b2),
        ],
        out_specs=row_spec(MUL * 4),
        out_shape=jax.ShapeDtypeStruct((n, MUL * 4), jnp.float32),
        scratch_shapes=[
            pltpu.VMEM((4 * MUL, MUL), jnp.bfloat16),
            pltpu.VMEM((6 * MUL, 3 * MUL), jnp.bfloat16),
        ],
        compiler_params=pltpu.CompilerParams(
            dimension_semantics=("arbitrary",),
            vmem_limit_bytes=60 * 1024 * 1024,
        ),
    )(x_0e, x1t, ytr, t4, w_ss, w_vv, w_sv, w_vs, b2)
